# Initial kernel scaffold; baseline (speedup 1.0000x reference)
import jax
import jax.numpy as jnp
from jax.experimental import pallas as pl


def kernel(ent_emb, rel_emb, time_emb, edge_index, b_rel, time_idx, inv, W_I, b_I, W_O, b_O, W_S, b_S, W_T, b_T):
    raise NotImplementedError("write your pallas kernel here")


# speedup vs baseline: 3.0010x; 3.0010x over previous
import functools

import jax
import jax.numpy as jnp
from jax import lax
from jax.experimental import pallas as pl
from jax.experimental.pallas import tpu as pltpu
from jax.experimental.pallas import tpu_sc as plsc

N = 10000
E = 320000
D = 128
R = 230
T = 366

RP = 240
TP = 368
NP = 10240
C = 32
G = 313
EP = 32 * C * G
ROWS_PER_TILE = NP // 16

ENT_BLK = 2000


def _tables_body(relp_ref, timep_ref, wrI_ref, wrO_ref, wtI_ref, wtO_ref,
                 wT_ref, bI_ref, bO_ref, bT_ref,
                 tabRI_ref, tabRO_ref, tabTI_ref, tabTO_ref, tnew_ref):
    r = relp_ref[...]
    t = timep_ref[...]
    tabRI_ref[...] = jnp.dot(r, wrI_ref[...], preferred_element_type=jnp.float32) + bI_ref[...]
    tabRO_ref[...] = jnp.dot(r, wrO_ref[...], preferred_element_type=jnp.float32) + bO_ref[...]
    tabTI_ref[...] = jnp.dot(t, wtI_ref[...], preferred_element_type=jnp.float32)
    tabTO_ref[...] = jnp.dot(t, wtO_ref[...], preferred_element_type=jnp.float32)
    tnew_ref[...] = jnp.dot(t, wT_ref[...], preferred_element_type=jnp.float32) + bT_ref[...]


def _tc_tables(relp, timep, wrI, wrO, wtI, wtO, wT, bI, bO, bT):
    f32 = jnp.float32
    return pl.pallas_call(
        _tables_body,
        out_shape=[
            jax.ShapeDtypeStruct((RP, D), f32),
            jax.ShapeDtypeStruct((RP, D), f32),
            jax.ShapeDtypeStruct((TP, D), f32),
            jax.ShapeDtypeStruct((TP, D), f32),
            jax.ShapeDtypeStruct((TP, D), f32),
        ],
    )(relp, timep, wrI, wrO, wtI, wtO, wT, bI, bO, bT)


def _ent_body(ent_ref, weI_ref, weO_ref, wS_ref, bS_ref,
              tabEI_ref, tabEO_ref, lin_ref):
    x = ent_ref[...]
    tabEI_ref[...] = jnp.dot(x, weI_ref[...], preferred_element_type=jnp.float32)
    tabEO_ref[...] = jnp.dot(x, weO_ref[...], preferred_element_type=jnp.float32)
    lin_ref[...] = jnp.dot(x, wS_ref[...], preferred_element_type=jnp.float32) + bS_ref[...]


def _tc_ent(ent_emb, weI, weO, wS, bS):
    f32 = jnp.float32
    nblk = N // ENT_BLK
    blk = lambda i: (i, 0)
    wblk = lambda i: (0, 0)
    return pl.pallas_call(
        _ent_body,
        grid=(nblk,),
        in_specs=[
            pl.BlockSpec((ENT_BLK, D), blk),
            pl.BlockSpec((D, D), wblk),
            pl.BlockSpec((D, D), wblk),
            pl.BlockSpec((D, D), wblk),
            pl.BlockSpec((1, D), wblk),
        ],
        out_specs=[
            pl.BlockSpec((ENT_BLK, D), blk),
            pl.BlockSpec((ENT_BLK, D), blk),
            pl.BlockSpec((ENT_BLK, D), blk),
        ],
        out_shape=[
            jax.ShapeDtypeStruct((N, D), f32),
            jax.ShapeDtypeStruct((N, D), f32),
            jax.ShapeDtypeStruct((N, D), f32),
        ],
    )(ent_emb, weI, weO, wS, bS)


def _sc_body(tabE, tabR, tabT, eidx, ridx, tidx, dst, rowids, zrows, ones_h,
             out, dout,
             acc, bufE, bufR, bufT, onesD,
             eidxv, ridxv, tidxv, dstv, rowv,
             semE, semR, semT):
    core = lax.axis_index("c")
    sub = lax.axis_index("s")
    rbase = sub * ROWS_PER_TILE
    obase = core * NP
    nrep = ROWS_PER_TILE // C
    ebase = (core * 16 + sub) * (G * C)

    pltpu.sync_copy(zrows, bufE)
    pltpu.sync_copy(ones_h, onesD)

    def _init(j, _):
        pltpu.sync_copy(rowids.at[pl.ds(rbase + j * C, C)], rowv)
        pltpu.sync_copy(bufE, acc.at[rowv])
        return 0

    lax.fori_loop(0, nrep, _init, 0)
    plsc.subcore_barrier()

    def _chunk(g, _):
        base = ebase + g * C
        pltpu.sync_copy(eidx.at[pl.ds(base, C)], eidxv)
        pltpu.sync_copy(ridx.at[pl.ds(base, C)], ridxv)
        pltpu.sync_copy(tidx.at[pl.ds(base, C)], tidxv)
        pltpu.sync_copy(dst.at[pl.ds(base, C)], dstv)
        cpE = pltpu.async_copy(tabE.at[eidxv], bufE, semE)
        cpR = pltpu.async_copy(tabR.at[ridxv], bufR, semR)
        cpT = pltpu.async_copy(tabT.at[tidxv], bufT, semT)
        cpE.wait()
        cpR.wait()
        cpT.wait()
        def _row(r, _):
            for k in range(8):
                sl = pl.ds(16 * k, 16)
                bufE[r, sl] = bufE[r, sl] + bufR[r, sl] + bufT[r, sl]
            return 0

        lax.fori_loop(0, C, _row, 0)
        pltpu.sync_copy(bufE, acc.at[dstv], add=True)
        return 0

    lax.fori_loop(0, G, _chunk, 0)
    plsc.subcore_barrier()

    def _writeout(j, _):
        pltpu.sync_copy(rowids.at[pl.ds(rbase + j * C, C)], rowv)
        pltpu.sync_copy(acc.at[rowv], bufE)
        pltpu.sync_copy(bufE, out.at[pl.ds(obase + rbase + j * C, C)])
        return 0

    lax.fori_loop(0, nrep, _writeout, 0)
    plsc.subcore_barrier()

    pltpu.sync_copy(zrows, bufR)

    def _init2(j, _):
        pltpu.sync_copy(rowids.at[pl.ds(rbase + j * C, C)], rowv)
        pltpu.sync_copy(bufR, acc.at[rowv])
        return 0

    lax.fori_loop(0, nrep, _init2, 0)
    plsc.subcore_barrier()

    def _chunk2(g, _):
        base = ebase + g * C
        pltpu.sync_copy(dst.at[pl.ds(base, C)], dstv)
        pltpu.sync_copy(onesD, acc.at[dstv], add=True)
        return 0

    lax.fori_loop(0, G, _chunk2, 0)
    plsc.subcore_barrier()

    def _writeout2(j, _):
        pltpu.sync_copy(rowids.at[pl.ds(rbase + j * C, C)], rowv)
        pltpu.sync_copy(acc.at[rowv], bufE)
        pltpu.sync_copy(bufE, dout.at[pl.ds(obase + rbase + j * C, C)])
        return 0

    lax.fori_loop(0, nrep, _writeout2, 0)


def _sc_scatter(tabE, tabR, tabT, eidx, ridx, tidx, dst, rowids, zrows, ones_h):
    f32 = jnp.float32
    i32 = jnp.int32
    mesh = plsc.VectorSubcoreMesh(core_axis_name="c", subcore_axis_name="s")
    fn = functools.partial(
        pl.kernel,
        out_type=[
            jax.ShapeDtypeStruct((2 * NP, D), f32),
            jax.ShapeDtypeStruct((2 * NP, D), f32),
        ],
        mesh=mesh,
        scratch_types=[
            pltpu.VMEM_SHARED((NP, D), f32),
            pltpu.VMEM((C, D), f32),
            pltpu.VMEM((C, D), f32),
            pltpu.VMEM((C, D), f32),
            pltpu.VMEM((C, D), f32),
            pltpu.VMEM((C,), i32),
            pltpu.VMEM((C,), i32),
            pltpu.VMEM((C,), i32),
            pltpu.VMEM((C,), i32),
            pltpu.VMEM((C,), i32),
            pltpu.SemaphoreType.DMA,
            pltpu.SemaphoreType.DMA,
            pltpu.SemaphoreType.DMA,
        ],
    )(_sc_body)
    return fn(tabE, tabR, tabT, eidx, ridx, tidx, dst, rowids, zrows, ones_h)


def _combine_body(p0_ref, p1_ref, d0_ref, d1_ref, lin_ref, out_ref):
    deg = jnp.maximum(d0_ref[...] + d1_ref[...], 1.0)
    out_ref[...] = lin_ref[...] + (p0_ref[...] + p1_ref[...]) / deg


def _tc_combine(p0, p1, d0, d1, lin):
    nblk = N // ENT_BLK
    blk = lambda i: (i, 0)
    return pl.pallas_call(
        _combine_body,
        grid=(nblk,),
        in_specs=[
            pl.BlockSpec((ENT_BLK, D), blk),
            pl.BlockSpec((ENT_BLK, D), blk),
            pl.BlockSpec((ENT_BLK, D), blk),
            pl.BlockSpec((ENT_BLK, D), blk),
            pl.BlockSpec((ENT_BLK, D), blk),
        ],
        out_specs=pl.BlockSpec((ENT_BLK, D), blk),
        out_shape=jax.ShapeDtypeStruct((N, D), jnp.float32),
    )(p0, p1, d0, d1, lin)


def kernel(ent_emb, rel_emb, time_emb, edge_index, b_rel, time_idx, inv,
           W_I, b_I, W_O, b_O, W_S, b_S, W_T, b_T):
    f32 = jnp.float32
    src = edge_index[0]
    dst = edge_index[1]

    pad = EP - E
    invp = jnp.pad(inv, (0, pad))
    eidx = jnp.pad(src, (0, pad)) + invp * N
    ridx = jnp.pad(b_rel, (0, pad)) + invp * RP
    tidx = jnp.pad(time_idx, (0, pad)) + invp * TP
    dstp = jnp.pad(dst, (0, pad), constant_values=N)

    relp = jnp.pad(rel_emb, ((0, RP - R), (0, 0)))
    timep = jnp.pad(time_emb, ((0, TP - T), (0, 0)))

    rowids = jnp.arange(NP, dtype=jnp.int32)
    zrows = jnp.zeros((C, D), f32)
    ones_h = jnp.ones((C, D), f32)

    bI2 = b_I.reshape(1, D)
    bO2 = b_O.reshape(1, D)
    bS2 = b_S.reshape(1, D)
    bT2 = b_T.reshape(1, D)

    tabRI, tabRO, tabTI, tabTO, time_new_p = _tc_tables(
        relp, timep,
        W_I[0:D, :], W_O[0:D, :], W_I[2 * D:3 * D, :], W_O[2 * D:3 * D, :],
        W_T, bI2, bO2, bT2)

    tabEI, tabEO, ent_lin = _tc_ent(
        ent_emb, W_I[D:2 * D, :], W_O[D:2 * D, :], W_S, bS2)

    tabE = jnp.concatenate([tabEI, tabEO], axis=0)
    tabR = jnp.concatenate([tabRI, tabRO], axis=0)
    tabT = jnp.concatenate([tabTI, tabTO], axis=0)

    partial, degp = _sc_scatter(tabE, tabR, tabT, eidx, ridx, tidx, dstp,
                                rowids, zrows, ones_h)

    ent_new = _tc_combine(partial[0:N], partial[NP:NP + N],
                          degp[0:N], degp[NP:NP + N], ent_lin)
    time_new = time_new_p[:T]
    return (ent_new, time_new)
